# baseline (speedup 1.0000x reference)
import functools

import jax
import jax.numpy as jnp
from jax import lax
from jax.experimental import pallas as pl
from jax.experimental.pallas import tpu as pltpu
from jax.experimental.pallas import tpu_sc as plsc

D_VOCAB = 30522
D_EMBED = 768
MAX_LEN = 512
N_SEG = 3
BATCH = 64
SEQ = 512

NC = 2
NS = 16
NW = NC * NS
P_PER_W = SEQ // NW
N_ROWS = BATCH * SEQ
LANES = 16
D_BLOCKS = D_EMBED // LANES
NBUF = 4
N_CHUNKS = BATCH


def _sc_body(tok_table, tokens_r, seg_r, pos_table, seg_table,
             out_hbm, tidx, sidx, bf0, bf1, bf2, bf3,
             psum, seg_v, smem_g,
             sg0, sg1, sg2, sg3, so0, so1, so2, so3):
  bufs = [bf0, bf1, bf2, bf3]
  sem_g = [sg0, sg1, sg2, sg3]
  sem_o = [so0, so1, so2, so3]

  cid = lax.axis_index("c")
  sid = lax.axis_index("s")
  wid = sid * NC + cid
  p0 = wid * P_PER_W

  pltpu.sync_copy(tokens_r.at[wid], tidx)
  pltpu.sync_copy(seg_r.at[wid], sidx)
  pltpu.sync_copy(seg_table, seg_v)
  pltpu.sync_copy(pos_table.at[pl.ds(p0, P_PER_W)], bf0)

  def prow(jj, _):
    for g in range(N_SEG):
      row = jj * N_SEG + g
      for d in range(D_BLOCKS):
        sl = pl.ds(d * LANES, LANES)
        psum[row, sl] = bf0[jj, sl] + seg_v[g, sl]
    return 0

  lax.fori_loop(0, P_PER_W, prow, 0)

  iota = lax.iota(jnp.int32, LANES)

  def stage_g(b, _):
    rowvec = N_SEG * iota + sidx[b, :]
    for jj in range(P_PER_W):
      smem_g[b * P_PER_W + jj] = rowvec[jj]
    return 0

  lax.fori_loop(0, BATCH, stage_g, 0)

  def gather_desc(b, k):
    return pltpu.make_async_copy(tok_table.at[tidx.at[b]], bufs[k], sem_g[k])

  def scatter_desc(b, k):
    return pltpu.make_async_copy(
        bufs[k], out_hbm.at[pl.ds(b * SEQ + p0, P_PER_W)], sem_o[k])

  gather_desc(0, 0).start()
  gather_desc(1, 1).start()

  def compute(b, k):
    buf = bufs[k]

    @plsc.parallel_loop(0, P_PER_W, step=1, unroll=4)
    def row_body(j):
      row = smem_g[b * P_PER_W + j]
      for d in range(D_BLOCKS):
        sl = pl.ds(d * LANES, LANES)
        plsc.addupdate(buf.at[j, sl], psum[row, sl])

  def quad(i, _):
    for k in range(NBUF):
      b = NBUF * i + k
      gather_desc(b, k).wait()
      k2 = (k + 2) % NBUF
      if k < 2:
        @pl.when(i > 0)
        def _():
          scatter_desc(b - 2, k2).wait()
        gather_desc(b + 2, k2).start()
      else:
        scatter_desc(b - 2, k2).wait()

        @pl.when(i < N_CHUNKS // NBUF - 1)
        def _():
          gather_desc(b + 2, k2).start()
      compute(b, k)
      scatter_desc(b, k).start()
    return 0

  lax.fori_loop(0, N_CHUNKS // NBUF, quad, 0)
  scatter_desc(N_CHUNKS - 2, 2).wait()
  scatter_desc(N_CHUNKS - 1, 3).wait()


@jax.jit
def _embed(tokens_r, seg_r, tok_table, pos_table, seg_table):
  mesh = plsc.VectorSubcoreMesh(core_axis_name="c", subcore_axis_name="s")
  f = functools.partial(
      pl.kernel,
      out_type=jax.ShapeDtypeStruct((N_ROWS, D_EMBED), jnp.float32),
      mesh=mesh,
      scratch_types=[
          pltpu.VMEM((BATCH, P_PER_W), jnp.int32),
          pltpu.VMEM((BATCH, P_PER_W), jnp.int32),
          pltpu.VMEM((P_PER_W, D_EMBED), jnp.float32),
          pltpu.VMEM((P_PER_W, D_EMBED), jnp.float32),
          pltpu.VMEM((P_PER_W, D_EMBED), jnp.float32),
          pltpu.VMEM((P_PER_W, D_EMBED), jnp.float32),
          pltpu.VMEM((P_PER_W * N_SEG, D_EMBED), jnp.float32),
          pltpu.VMEM((N_SEG, D_EMBED), jnp.float32),
          pltpu.SMEM((BATCH * P_PER_W,), jnp.int32),
          pltpu.SemaphoreType.DMA,
          pltpu.SemaphoreType.DMA,
          pltpu.SemaphoreType.DMA,
          pltpu.SemaphoreType.DMA,
          pltpu.SemaphoreType.DMA,
          pltpu.SemaphoreType.DMA,
          pltpu.SemaphoreType.DMA,
          pltpu.SemaphoreType.DMA,
      ],
  )(_sc_body)
  return f(tok_table, tokens_r, seg_r, pos_table, seg_table)


def _rearrange_ids(x):
  return x.reshape(BATCH, NW, P_PER_W).transpose(1, 0, 2)


def kernel(tokens, segment_label, tok_table, pos_table, seg_table):
  out = _embed(_rearrange_ids(tokens), _rearrange_ids(segment_label),
               tok_table, pos_table, seg_table)
  return out.reshape(BATCH, SEQ, D_EMBED)


# speedup vs baseline: 1.2196x; 1.2196x over previous
import functools

import jax
import jax.numpy as jnp
from jax import lax
from jax.experimental import pallas as pl
from jax.experimental.pallas import tpu as pltpu
from jax.experimental.pallas import tpu_sc as plsc

D_VOCAB = 30522
D_EMBED = 768
MAX_LEN = 512
N_SEG = 3
BATCH = 64
SEQ = 512

NC = 2
NS = 16
NW = NC * NS
P_PER_W = SEQ // NW
N_ROWS = BATCH * SEQ
LANES = 16
D_BLOCKS = D_EMBED // LANES
NBUF = 6
LOOKAHEAD = 3
N_CHUNKS = BATCH


def _sc_body(tok_table, tokens_r, seg_r, pos_table, seg_table,
             out_hbm, tidx, sidx, bf0, bf1, bf2, bf3, bf4, bf5,
             psum, seg_v, smem_g,
             sg0, sg1, sg2, sg3, sg4, sg5, so0, so1, so2, so3, so4, so5):
  bufs = [bf0, bf1, bf2, bf3, bf4, bf5]
  sem_g = [sg0, sg1, sg2, sg3, sg4, sg5]
  sem_o = [so0, so1, so2, so3, so4, so5]

  cid = lax.axis_index("c")
  sid = lax.axis_index("s")
  wid = sid * NC + cid
  p0 = wid * P_PER_W

  pltpu.sync_copy(tokens_r.at[wid], tidx)
  pltpu.sync_copy(seg_r.at[wid], sidx)
  pltpu.sync_copy(seg_table, seg_v)
  pltpu.sync_copy(pos_table.at[pl.ds(p0, P_PER_W)], bf0)

  def prow(jj, _):
    for g in range(N_SEG):
      row = jj * N_SEG + g
      for d in range(D_BLOCKS):
        sl = pl.ds(d * LANES, LANES)
        psum[row, sl] = bf0[jj, sl] + seg_v[g, sl]
    return 0

  lax.fori_loop(0, P_PER_W, prow, 0)

  iota = lax.iota(jnp.int32, LANES)

  def stage_g(b, _):
    rowvec = N_SEG * iota + sidx[b, :]
    for jj in range(P_PER_W):
      smem_g[b * P_PER_W + jj] = rowvec[jj]
    return 0

  lax.fori_loop(0, BATCH, stage_g, 0)

  def gather_desc(b, k):
    return pltpu.make_async_copy(tok_table.at[tidx.at[b]], bufs[k], sem_g[k])

  def scatter_desc(b, k):
    return pltpu.make_async_copy(
        bufs[k], out_hbm.at[pl.ds(b * SEQ + p0, P_PER_W)], sem_o[k])

  for b0 in range(LOOKAHEAD):
    gather_desc(b0, b0).start()

  def compute(b, k):
    buf = bufs[k]

    @plsc.parallel_loop(0, P_PER_W, step=1, unroll=2)
    def row_body(j):
      row = smem_g[b * P_PER_W + j]
      for d in range(D_BLOCKS):
        sl = pl.ds(d * LANES, LANES)
        plsc.addupdate(buf.at[j, sl], psum[row, sl])

  def phase(b, k, wait_scatter_pred, do_gather):
    gather_desc(b, k).wait()
    k3 = (k + LOOKAHEAD) % NBUF
    if wait_scatter_pred is None:
      scatter_desc(b - LOOKAHEAD, k3).wait()
    else:
      @pl.when(wait_scatter_pred)
      def _():
        scatter_desc(b - LOOKAHEAD, k3).wait()
    if do_gather:
      gather_desc(b + LOOKAHEAD, k3).start()
    compute(b, k)
    scatter_desc(b, k).start()

  n_trips = N_CHUNKS // NBUF

  def trip_body(i, _):
    for k in range(NBUF):
      b = NBUF * i + k
      pred = (i > 0) if k < LOOKAHEAD else None
      phase(b, k, pred, True)
    return 0

  lax.fori_loop(0, n_trips, trip_body, 0)

  for b in range(n_trips * NBUF, N_CHUNKS):
    phase(b, b % NBUF, None, b + LOOKAHEAD < N_CHUNKS)
  for b in range(N_CHUNKS - LOOKAHEAD, N_CHUNKS):
    scatter_desc(b, b % NBUF).wait()


@jax.jit
def _embed(tokens_r, seg_r, tok_table, pos_table, seg_table):
  mesh = plsc.VectorSubcoreMesh(core_axis_name="c", subcore_axis_name="s")
  f = functools.partial(
      pl.kernel,
      out_type=jax.ShapeDtypeStruct((N_ROWS, D_EMBED), jnp.float32),
      mesh=mesh,
      scratch_types=[
          pltpu.VMEM((BATCH, P_PER_W), jnp.int32),
          pltpu.VMEM((BATCH, P_PER_W), jnp.int32),
          pltpu.VMEM((P_PER_W, D_EMBED), jnp.float32),
          pltpu.VMEM((P_PER_W, D_EMBED), jnp.float32),
          pltpu.VMEM((P_PER_W, D_EMBED), jnp.float32),
          pltpu.VMEM((P_PER_W, D_EMBED), jnp.float32),
          pltpu.VMEM((P_PER_W, D_EMBED), jnp.float32),
          pltpu.VMEM((P_PER_W, D_EMBED), jnp.float32),
          pltpu.VMEM((P_PER_W * N_SEG, D_EMBED), jnp.float32),
          pltpu.VMEM((N_SEG, D_EMBED), jnp.float32),
          pltpu.SMEM((BATCH * P_PER_W,), jnp.int32),
      ] + [pltpu.SemaphoreType.DMA] * 12,
  )(_sc_body)
  return f(tok_table, tokens_r, seg_r, pos_table, seg_table)


def _rearrange_ids(x):
  return x.reshape(BATCH, NW, P_PER_W).transpose(1, 0, 2)


def kernel(tokens, segment_label, tok_table, pos_table, seg_table):
  out = _embed(_rearrange_ids(tokens), _rearrange_ids(segment_label),
               tok_table, pos_table, seg_table)
  return out.reshape(BATCH, SEQ, D_EMBED)


# speedup vs baseline: 1.3074x; 1.0720x over previous
import functools

import jax
import jax.numpy as jnp
from jax import lax
from jax.experimental import pallas as pl
from jax.experimental.pallas import tpu as pltpu
from jax.experimental.pallas import tpu_sc as plsc

D_VOCAB = 30522
D_EMBED = 768
MAX_LEN = 512
N_SEG = 3
BATCH = 64
SEQ = 512

NC = 2
NS = 16
NW = NC * NS
P_PER_W = SEQ // NW
N_ROWS = BATCH * SEQ
LANES = 16
D_BLOCKS = D_EMBED // LANES
NBUF = 4
N_CHUNKS = BATCH


def _sc_body(tok_table, tokens_r, seg_r, pos_table, seg_table,
             out_hbm, tidx, sidx, bf0, bf1, bf2, bf3,
             psum, seg_v, smem_g,
             sg0, sg1, sg2, sg3, so0, so1, so2, so3):
  bufs = [bf0, bf1, bf2, bf3]
  sem_g = [sg0, sg1, sg2, sg3]
  sem_o = [so0, so1, so2, so3]

  cid = lax.axis_index("c")
  sid = lax.axis_index("s")
  wid = sid * NC + cid
  p0 = wid * P_PER_W

  pltpu.sync_copy(tokens_r.at[wid], tidx)
  pltpu.sync_copy(seg_r.at[wid], sidx)

  pltpu.make_async_copy(tok_table.at[tidx.at[0]], bf0, sg0).start()
  pltpu.make_async_copy(tok_table.at[tidx.at[1]], bf1, sg1).start()

  pltpu.sync_copy(seg_table, seg_v)
  pltpu.sync_copy(pos_table.at[pl.ds(p0, P_PER_W)], bf3)

  def prow(jj, _):
    for g in range(N_SEG):
      row = jj * N_SEG + g
      for d in range(D_BLOCKS):
        sl = pl.ds(d * LANES, LANES)
        psum[row, sl] = bf3[jj, sl] + seg_v[g, sl]
    return 0

  lax.fori_loop(0, P_PER_W, prow, 0)

  iota = lax.iota(jnp.int32, LANES)

  def stage_g(b, _):
    rowvec = N_SEG * iota + sidx[b, :]
    for jj in range(P_PER_W):
      smem_g[b * P_PER_W + jj] = rowvec[jj]
    return 0

  lax.fori_loop(0, BATCH, stage_g, 0)

  def gather_desc(b, k):
    return pltpu.make_async_copy(tok_table.at[tidx.at[b]], bufs[k], sem_g[k])

  def scatter_desc(b, k):
    return pltpu.make_async_copy(
        bufs[k], out_hbm.at[pl.ds(b * SEQ + p0, P_PER_W)], sem_o[k])

  def compute(b, k):
    buf = bufs[k]

    @plsc.parallel_loop(0, P_PER_W, step=1, unroll=2)
    def row_body(j):
      row = smem_g[b * P_PER_W + j]
      for d in range(D_BLOCKS):
        sl = pl.ds(d * LANES, LANES)
        plsc.addupdate(buf.at[j, sl], psum[row, sl])

  def quad(i, _):
    for k in range(NBUF):
      b = NBUF * i + k
      gather_desc(b, k).wait()
      k2 = (k + 2) % NBUF
      if k < 2:
        @pl.when(i > 0)
        def _():
          scatter_desc(b - 2, k2).wait()
        gather_desc(b + 2, k2).start()
      else:
        scatter_desc(b - 2, k2).wait()

        @pl.when(i < N_CHUNKS // NBUF - 1)
        def _():
          gather_desc(b + 2, k2).start()
      compute(b, k)
      scatter_desc(b, k).start()
    return 0

  lax.fori_loop(0, N_CHUNKS // NBUF, quad, 0)
  scatter_desc(N_CHUNKS - 2, 2).wait()
  scatter_desc(N_CHUNKS - 1, 3).wait()


@jax.jit
def _embed(tokens_r, seg_r, tok_table, pos_table, seg_table):
  mesh = plsc.VectorSubcoreMesh(core_axis_name="c", subcore_axis_name="s")
  f = functools.partial(
      pl.kernel,
      out_type=jax.ShapeDtypeStruct((N_ROWS, D_EMBED), jnp.float32),
      mesh=mesh,
      scratch_types=[
          pltpu.VMEM((BATCH, P_PER_W), jnp.int32),
          pltpu.VMEM((BATCH, P_PER_W), jnp.int32),
          pltpu.VMEM((P_PER_W, D_EMBED), jnp.float32),
          pltpu.VMEM((P_PER_W, D_EMBED), jnp.float32),
          pltpu.VMEM((P_PER_W, D_EMBED), jnp.float32),
          pltpu.VMEM((P_PER_W, D_EMBED), jnp.float32),
          pltpu.VMEM((P_PER_W * N_SEG, D_EMBED), jnp.float32),
          pltpu.VMEM((N_SEG, D_EMBED), jnp.float32),
          pltpu.SMEM((BATCH * P_PER_W,), jnp.int32),
          pltpu.SemaphoreType.DMA,
          pltpu.SemaphoreType.DMA,
          pltpu.SemaphoreType.DMA,
          pltpu.SemaphoreType.DMA,
          pltpu.SemaphoreType.DMA,
          pltpu.SemaphoreType.DMA,
          pltpu.SemaphoreType.DMA,
          pltpu.SemaphoreType.DMA,
      ],
  )(_sc_body)
  return f(tok_table, tokens_r, seg_r, pos_table, seg_table)


def _rearrange_ids(x):
  return x.reshape(BATCH, NW, P_PER_W).transpose(1, 0, 2)


def kernel(tokens, segment_label, tok_table, pos_table, seg_table):
  out = _embed(_rearrange_ids(tokens), _rearrange_ids(segment_label),
               tok_table, pos_table, seg_table)
  return out.reshape(BATCH, SEQ, D_EMBED)
